# baseline (speedup 1.0000x reference)
import functools

import jax
import jax.numpy as jnp
from jax import lax
from jax.experimental import pallas as pl
from jax.experimental.pallas import tpu as pltpu
from jax.experimental.pallas import tpu_sc as plsc

HIDDEN = 64
WINDOW = 128


def kernel(input_ids, W):
    B, S = input_ids.shape
    n = B * S
    idx = input_ids.reshape(1, n).astype(jnp.int32)
    table2 = W.reshape(W.shape[0] // 2, 2 * HIDDEN)

    mesh = plsc.VectorSubcoreMesh(core_axis_name="core",
                                  subcore_axis_name="subcore")

    @functools.partial(
        pl.kernel,
        out_type=jax.ShapeDtypeStruct((n, HIDDEN), jnp.float32),
        mesh=mesh,
        scratch_types=[
            pltpu.VMEM((WINDOW,), jnp.int32),
            pltpu.VMEM((WINDOW, 2 * HIDDEN), jnp.float32),
        ],
    )
    def gather_kernel(table_hbm, idx_hbm, out_hbm, pidx_v, pair_v):
        def body(i_vmem, o_vmem):
            @pl.loop(0, WINDOW, step=16)
            def _(c):
                v = i_vmem[0, pl.ds(c, 16)]
                pidx_v[pl.ds(c, 16)] = lax.shift_right_logical(v, 1)

            pltpu.sync_copy(table_hbm.at[pidx_v], pair_v)

            @pl.loop(0, WINDOW, step=16)
            def _(g):
                hv = (i_vmem[0, pl.ds(g, 16)] & 1) * HIDDEN
                for j in range(16):
                    h = hv[j]
                    for k in range(HIDDEN // 16):
                        o_vmem[g + j, pl.ds(16 * k, 16)] = (
                            pair_v[g + j, pl.ds(h + 16 * k, 16)])

        pltpu.emit_pipeline(
            body,
            grid=(n // WINDOW,),
            in_specs=[pl.BlockSpec((1, WINDOW), lambda i: (0, i))],
            out_specs=[pl.BlockSpec((WINDOW, HIDDEN), lambda i: (i, 0))],
            core_axis_name=("core", "subcore"),
            dimension_semantics=(pltpu.PARALLEL,),
        )(idx_hbm, out_hbm)

    out = gather_kernel(table2, idx)
    return out.reshape(B, S, HIDDEN)


# speedup vs baseline: 1.1870x; 1.1870x over previous
import functools

import jax
import jax.numpy as jnp
from jax import lax
from jax.experimental import pallas as pl
from jax.experimental.pallas import tpu as pltpu
from jax.experimental.pallas import tpu_sc as plsc

HIDDEN = 64
SEQ = 20
NUM_WORKERS = 32
ROWS_PER_WIN = 32
TOK_PER_WIN = ROWS_PER_WIN * SEQ
CHUNK = 128
CHUNKS_PER_WIN = TOK_PER_WIN // CHUNK


def kernel(input_ids, W):
    B, S = input_ids.shape
    n = B * S
    n_wins = B // (NUM_WORKERS * ROWS_PER_WIN)
    idx = input_ids.reshape(n).astype(jnp.int32)
    table2 = W.reshape(W.shape[0] // 2, 2 * HIDDEN)

    mesh = plsc.VectorSubcoreMesh(core_axis_name="core",
                                  subcore_axis_name="subcore")

    @functools.partial(
        pl.kernel,
        out_type=jax.ShapeDtypeStruct((B, S, HIDDEN), jnp.float32),
        mesh=mesh,
        scratch_types=[
            pltpu.VMEM((TOK_PER_WIN,), jnp.int32),
            pltpu.VMEM((TOK_PER_WIN,), jnp.int32),
            pltpu.VMEM((TOK_PER_WIN,), jnp.int32),
            pltpu.VMEM((TOK_PER_WIN,), jnp.int32),
            pltpu.VMEM((TOK_PER_WIN,), jnp.int32),
            pltpu.VMEM((TOK_PER_WIN,), jnp.int32),
            pltpu.VMEM((2, CHUNK, 2 * HIDDEN), jnp.float32),
            pltpu.VMEM((TOK_PER_WIN, HIDDEN), jnp.float32),
            pltpu.SemaphoreType.DMA,
            pltpu.SemaphoreType.DMA,
            pltpu.SemaphoreType.DMA,
        ],
    )
    def gather_kernel(table_hbm, idx_hbm, out_hbm,
                      ibuf0, ibuf1, pbuf0, pbuf1, hbuf0, hbuf1,
                      pair, obuf, isem, gsem, osem):
        ibufs, pbufs, hbufs = (ibuf0, ibuf1), (pbuf0, pbuf1), (hbuf0, hbuf1)
        wid = lax.axis_index("subcore") * 2 + lax.axis_index("core")
        base_row = wid * (n_wins * ROWS_PER_WIN)
        base_tok = base_row * SEQ

        def idx_copy(win, slot):
            return pltpu.make_async_copy(
                idx_hbm.at[pl.ds(base_tok + win * TOK_PER_WIN, TOK_PER_WIN)],
                ibufs[slot], isem)

        def gather_copy(c, islot, pslot):
            return pltpu.make_async_copy(
                table_hbm.at[pbufs[islot].at[pl.ds(c * CHUNK, CHUNK)]],
                pair.at[pslot], gsem)

        def out_copy(win):
            return pltpu.make_async_copy(
                obuf.reshape(ROWS_PER_WIN, SEQ, HIDDEN),
                out_hbm.at[pl.ds(base_row + win * ROWS_PER_WIN, ROWS_PER_WIN)],
                osem)

        def repack(slot):
            @pl.loop(0, TOK_PER_WIN, step=16)
            def _(g):
                v = ibufs[slot][pl.ds(g, 16)]
                pbufs[slot][pl.ds(g, 16)] = lax.shift_right_logical(v, 1)
                hbufs[slot][pl.ds(g, 16)] = (v & 1) * HIDDEN

        def extract(c, slot, pslot):
            @pl.loop(0, CHUNK, step=16)
            def _(g):
                hv = hbufs[slot][pl.ds(c * CHUNK + g, 16)]
                for j in range(16):
                    t = c * CHUNK + g + j
                    h = hv[j]
                    for k in range(HIDDEN // 16):
                        obuf[t, pl.ds(16 * k, 16)] = (
                            pair[pslot, g + j, pl.ds(h + 16 * k, 16)])

        def window(win, slot):
            idx_copy(win, slot).wait()

            @pl.when(win + 1 < n_wins)
            def _():
                idx_copy(win + 1, 1 - slot).start()

            repack(slot)
            gather_copy(0, slot, 0).start()

            @pl.when(win >= 1)
            def _():
                out_copy(win - 1).wait()

            for c in range(CHUNKS_PER_WIN):
                p = c % 2
                if c + 1 < CHUNKS_PER_WIN:
                    gather_copy(c + 1, slot, 1 - p).start()
                gather_copy(c, slot, p).wait()
                extract(c, slot, p)

            out_copy(win).start()

        idx_copy(0, 0).start()

        @pl.loop(0, n_wins, step=2)
        def _(win):
            window(win, 0)
            window(win + 1, 1)

        out_copy(n_wins - 1).wait()

    out = gather_kernel(table2, idx)
    return out
